# Initial kernel scaffold; baseline (speedup 1.0000x reference)
import functools

import jax
import jax.numpy as jnp
from jax import lax
from jax.experimental import pallas as pl
from jax.experimental.pallas import tpu as pltpu
from jax.experimental.pallas import tpu_sc as plsc

N_NODES = 10000
N_EDGES = 160000

NC = 2
NS = 16
NW = NC * NS
NPW = 313
NPAD = NW * NPW

K_FLUSH = 4096
G = 128
ECAP = N_EDGES + K_FLUSH + 256
SCH = 8000
BUFCAP = K_FLUSH + 192

_NEG_INF = float("-inf")


def _worker_id():
    return lax.axis_index("s") * NC + lax.axis_index("c")


def _sc_mesh():
    return plsc.VectorSubcoreMesh(
        core_axis_name="c", subcore_axis_name="s",
        num_cores=NC, num_subcores=NS)


@functools.partial(
    pl.kernel,
    out_type=[
        jax.ShapeDtypeStruct((NW, ECAP), jnp.int32),
        jax.ShapeDtypeStruct((NW, ECAP), jnp.int32),
        jax.ShapeDtypeStruct((NW, 16), jnp.int32),
    ],
    mesh=_sc_mesh(),
    scratch_types=[
        pltpu.VMEM((SCH,), jnp.int32),
        pltpu.VMEM((SCH,), jnp.int32),
        pltpu.VMEM((BUFCAP,), jnp.int32),
        pltpu.VMEM((BUFCAP,), jnp.int32),
        pltpu.VMEM((16,), jnp.int32),
    ],
)
def _bin_edges(src_hbm, dst_hbm, bsrc_hbm, bloc_hbm, cnt_hbm,
               stage_s, stage_d, buf_s, buf_l, cnt_v):
    w = _worker_id()
    lo = w * NPW

    def do_flush(pos, flushed):
        pltpu.sync_copy(buf_s.at[pl.ds(0, K_FLUSH)],
                        bsrc_hbm.at[w, pl.ds(flushed, K_FLUSH)])
        pltpu.sync_copy(buf_l.at[pl.ds(0, K_FLUSH)],
                        bloc_hbm.at[w, pl.ds(flushed, K_FLUSH)])
        ts = buf_s[pl.ds(K_FLUSH, 16)]
        tl = buf_l[pl.ds(K_FLUSH, 16)]
        buf_s[pl.ds(0, 16)] = ts
        buf_l[pl.ds(0, 16)] = tl
        return pos - K_FLUSH, flushed + K_FLUSH

    def no_flush(pos, flushed):
        return pos, flushed

    def append_chunk(i, carry):
        pos, flushed = carry
        d = stage_d[pl.ds(i * 16, 16)]
        s = stage_s[pl.ds(i * 16, 16)]
        m = (d >= lo) & (d < lo + NPW)
        plsc.store_compressed(buf_s.at[pl.ds(pos, 16)], s, mask=m)
        plsc.store_compressed(buf_l.at[pl.ds(pos, 16)], d - lo, mask=m)
        c = plsc.all_reduce_population_count(m)
        pos = pos + c[0]
        return lax.cond(pos >= K_FLUSH, do_flush, no_flush, pos, flushed)

    def stage_loop(cb, carry):
        pltpu.sync_copy(src_hbm.at[pl.ds(cb * SCH, SCH)], stage_s)
        pltpu.sync_copy(dst_hbm.at[pl.ds(cb * SCH, SCH)], stage_d)
        return lax.fori_loop(0, SCH // 16, append_chunk, carry)

    pos, flushed = lax.fori_loop(0, N_EDGES // SCH, stage_loop,
                                 (jnp.int32(0), jnp.int32(0)))
    n_total = flushed + pos

    zeros16 = jnp.zeros((16,), jnp.int32)
    pad16 = jnp.full((16,), NPW, jnp.int32)
    for j in range(G // 16):
        buf_s[pl.ds(pos + j * 16, 16)] = zeros16
        buf_l[pl.ds(pos + j * 16, 16)] = pad16
    pos = pos + G
    pos, flushed = lax.cond(pos >= K_FLUSH, do_flush, no_flush, pos, flushed)

    pltpu.sync_copy(buf_s.at[pl.ds(0, K_FLUSH)],
                    bsrc_hbm.at[w, pl.ds(flushed, K_FLUSH)])
    pltpu.sync_copy(buf_l.at[pl.ds(0, K_FLUSH)],
                    bloc_hbm.at[w, pl.ds(flushed, K_FLUSH)])

    cnt_v[pl.ds(0, 16)] = jnp.full((16,), 0, jnp.int32) + n_total
    pltpu.sync_copy(cnt_v, cnt_hbm.at[w])


def _make_segmax(C):
    @functools.partial(
        pl.kernel,
        out_type=jax.ShapeDtypeStruct((NPAD, C), jnp.float32),
        mesh=_sc_mesh(),
        scratch_types=[
            pltpu.VMEM((NPW + 1, C), jnp.float32),
            pltpu.VMEM((G,), jnp.int32),
            pltpu.VMEM((G,), jnp.int32),
            pltpu.VMEM((G, C), jnp.float32),
            pltpu.VMEM((16,), jnp.int32),
            pltpu.SemaphoreType.DMA,
        ],
    )
    def seg_kernel(q_hbm, bsrc_hbm, bloc_hbm, cnt_hbm, s_hbm,
                   acc, idx, loc, rows, cnt_v, sem):
        w = _worker_id()
        lo = w * NPW
        pltpu.sync_copy(cnt_hbm.at[w], cnt_v)
        n = cnt_v[0]
        nchunks = (n + (G - 1)) // G

        neg = jnp.full((16,), _NEG_INF, jnp.float32)

        def init_body(i, _):
            for j in range(C // 16):
                acc[i, pl.ds(j * 16, 16)] = neg
            return 0
        lax.fori_loop(0, NPW + 1, init_body, 0)

        def chunk_body(cb, _):
            base = cb * G
            pltpu.sync_copy(bsrc_hbm.at[w, pl.ds(base, G)], idx)
            pltpu.sync_copy(bloc_hbm.at[w, pl.ds(base, G)], loc)
            pltpu.async_copy(q_hbm.at[idx], rows, sem).wait()

            def edge_body(i, _):
                r = loc[i]
                for j in range(C // 16):
                    sl = pl.ds(j * 16, 16)
                    acc[r, sl] = jnp.maximum(acc[r, sl], rows[i, sl])
                return 0
            lax.fori_loop(0, G, edge_body, 0)
            return 0
        lax.fori_loop(0, nchunks, chunk_body, 0)

        pltpu.sync_copy(acc.at[pl.ds(0, NPW)], s_hbm.at[pl.ds(lo, NPW)])

    return seg_kernel


_seg64 = _make_segmax(64)
_seg128 = _make_segmax(128)
_seg256 = _make_segmax(256)


_TR = 1000


def _tc_first(x, A, bias, C):
    cin = x.shape[1]

    def body(x_ref, a_ref, b_ref, p_ref, q_ref):
        r = jnp.dot(x_ref[...], a_ref[...],
                    preferred_element_type=jnp.float32) + b_ref[...]
        p_ref[...] = r[:, :C]
        q_ref[...] = r[:, C:]

    return pl.pallas_call(
        body,
        grid=(N_NODES // _TR,),
        in_specs=[
            pl.BlockSpec((_TR, cin), lambda i: (i, 0)),
            pl.BlockSpec((cin, 2 * C), lambda i: (0, 0)),
            pl.BlockSpec((1, 2 * C), lambda i: (0, 0)),
        ],
        out_specs=[
            pl.BlockSpec((_TR, C), lambda i: (i, 0)),
            pl.BlockSpec((_TR, C), lambda i: (i, 0)),
        ],
        out_shape=[jax.ShapeDtypeStruct((N_NODES, C), jnp.float32)] * 2,
    )(x, A, bias)


def _tc_mid(p_prev, s_prev, A, bias, C):
    cin = p_prev.shape[1]

    def body(p_ref, s_ref, a_ref, b_ref, po_ref, qo_ref):
        xv = jnp.maximum(p_ref[...] + s_ref[...], 0.0)
        r = jnp.dot(xv, a_ref[...],
                    preferred_element_type=jnp.float32) + b_ref[...]
        po_ref[...] = r[:, :C]
        qo_ref[...] = r[:, C:]

    return pl.pallas_call(
        body,
        grid=(N_NODES // _TR,),
        in_specs=[
            pl.BlockSpec((_TR, cin), lambda i: (i, 0)),
            pl.BlockSpec((_TR, cin), lambda i: (i, 0)),
            pl.BlockSpec((cin, 2 * C), lambda i: (0, 0)),
            pl.BlockSpec((1, 2 * C), lambda i: (0, 0)),
        ],
        out_specs=[
            pl.BlockSpec((_TR, C), lambda i: (i, 0)),
            pl.BlockSpec((_TR, C), lambda i: (i, 0)),
        ],
        out_shape=[jax.ShapeDtypeStruct((N_NODES, C), jnp.float32)] * 2,
    )(p_prev, s_prev, A, bias)


def _tc_final(p3, s3a, s3b, x0, W4, b4, W5, b5):
    def body(p_ref, sa_ref, sb_ref, x0_ref, w4_ref, b4_ref, w5_ref, b5_ref,
             o_ref):
        s = jnp.concatenate([sa_ref[...], sb_ref[...]], axis=1)
        xv = jnp.maximum(p_ref[...] + s, 0.0)
        h = jnp.maximum(
            jnp.dot(xv, w4_ref[...], preferred_element_type=jnp.float32)
            + b4_ref[...], 0.0)
        o_ref[...] = (jnp.dot(h, w5_ref[...],
                              preferred_element_type=jnp.float32)
                      + b5_ref[...] + x0_ref[...])

    return pl.pallas_call(
        body,
        grid=(N_NODES // _TR,),
        in_specs=[
            pl.BlockSpec((_TR, 512), lambda i: (i, 0)),
            pl.BlockSpec((_TR, 256), lambda i: (i, 0)),
            pl.BlockSpec((_TR, 256), lambda i: (i, 0)),
            pl.BlockSpec((_TR, 3), lambda i: (i, 0)),
            pl.BlockSpec((512, 256), lambda i: (0, 0)),
            pl.BlockSpec((1, 256), lambda i: (0, 0)),
            pl.BlockSpec((256, 3), lambda i: (0, 0)),
            pl.BlockSpec((1, 3), lambda i: (0, 0)),
        ],
        out_specs=pl.BlockSpec((_TR, 3), lambda i: (i, 0)),
        out_shape=jax.ShapeDtypeStruct((N_NODES, 3), jnp.float32),
    )(p3, s3a, s3b, x0, W4, b4, W5, b5)


def _split_weights(W, b, cin):
    wa, wb = W[:cin], W[cin:]
    A = jnp.concatenate([wa - wb, wb], axis=1)
    bias = jnp.concatenate([b, jnp.zeros_like(b)])[None, :]
    return A, bias


def kernel(x, edge_index, W1, b1, W2, b2, W3, b3, W4, b4, W5, b5):
    src = edge_index[0]
    dst = edge_index[1]

    bsrc, bloc, counts = _bin_edges(src, dst)

    A1, bias1 = _split_weights(W1, b1, 3)
    A2, bias2 = _split_weights(W2, b2, 64)
    A3, bias3 = _split_weights(W3, b3, 128)

    P1, Q1 = _tc_first(x, A1, bias1, 64)
    S1 = _seg64(Q1, bsrc, bloc, counts)[:N_NODES]

    P2, Q2 = _tc_mid(P1, S1, A2, bias2, 128)
    S2 = _seg128(Q2, bsrc, bloc, counts)[:N_NODES]

    P3, Q3 = _tc_mid(P2, S2, A3, bias3, 512)
    S3a = _seg256(Q3[:, :256], bsrc, bloc, counts)[:N_NODES]
    S3b = _seg256(Q3[:, 256:], bsrc, bloc, counts)[:N_NODES]

    return _tc_final(P3, S3a, S3b, x, W4, b4[None, :], W5, b5[None, :])


# speedup vs baseline: 2.0413x; 2.0413x over previous
import functools

import jax
import jax.numpy as jnp
from jax import lax
from jax.experimental import pallas as pl
from jax.experimental.pallas import tpu as pltpu
from jax.experimental.pallas import tpu_sc as plsc

N_NODES = 10000
N_EDGES = 160000

NC = 2
NS = 16
NW = NC * NS
NPW = 320
NPAD = NW * NPW

K_FLUSH = 4096
G = 128
ECAP = N_EDGES + K_FLUSH + 256
SCH = 8000
BUFCAP = K_FLUSH + 192

_NEG_INF = float("-inf")


def _worker_id():
    return lax.axis_index("s") * NC + lax.axis_index("c")


def _sc_mesh():
    return plsc.VectorSubcoreMesh(
        core_axis_name="c", subcore_axis_name="s",
        num_cores=NC, num_subcores=NS)


@functools.lru_cache(maxsize=None)
def _get_bin_kernel():
    @functools.partial(
        pl.kernel,
        out_type=[
            jax.ShapeDtypeStruct((NW * ECAP,), jnp.int32),
            jax.ShapeDtypeStruct((NW * ECAP,), jnp.int32),
            jax.ShapeDtypeStruct((NW * 16,), jnp.int32),
        ],
        mesh=_sc_mesh(),
        scratch_types=[
            pltpu.VMEM((SCH,), jnp.int32),
            pltpu.VMEM((SCH,), jnp.int32),
            pltpu.VMEM((BUFCAP,), jnp.int32),
            pltpu.VMEM((BUFCAP,), jnp.int32),
            pltpu.VMEM((16,), jnp.int32),
        ],
        compiler_params=pltpu.CompilerParams(needs_layout_passes=False),
    )
    def bin_edges(src_hbm, dst_hbm, bsrc_hbm, bloc_hbm, cnt_hbm,
                  stage_s, stage_d, buf_s, buf_l, cnt_v):
        w = _worker_id()
        lo = w * NPW

        def do_flush(pos, flushed):
            off = pl.multiple_of(w * ECAP + flushed, 8)
            pltpu.sync_copy(buf_s.at[pl.ds(0, K_FLUSH)],
                            bsrc_hbm.at[pl.ds(off, K_FLUSH)])
            pltpu.sync_copy(buf_l.at[pl.ds(0, K_FLUSH)],
                            bloc_hbm.at[pl.ds(off, K_FLUSH)])
            ts = buf_s[pl.ds(K_FLUSH, 16)]
            tl = buf_l[pl.ds(K_FLUSH, 16)]
            buf_s[pl.ds(0, 16)] = ts
            buf_l[pl.ds(0, 16)] = tl
            return pos - K_FLUSH, flushed + K_FLUSH

        def no_flush(pos, flushed):
            return pos, flushed

        lo_v = jnp.full((16,), lo, jnp.int32)
        hi_v = jnp.full((16,), lo + NPW, jnp.int32)
        zero_v = jnp.zeros((16,), jnp.int32)
        one_v = jnp.full((16,), 1, jnp.int32)

        def append_chunk(i, carry):
            pos, flushed = carry
            d = stage_d[pl.ds(i * 16, 16)]
            s = stage_s[pl.ds(i * 16, 16)]
            m = (d >= lo_v) & (d < hi_v)
            csum = plsc.cumsum(jnp.where(m, one_v, zero_v))
            pos_v = jnp.full((16,), pos, jnp.int32)
            idxv = jnp.maximum(pos_v + csum - one_v, zero_v)
            plsc.store_scatter(buf_s, [idxv], s, mask=m)
            plsc.store_scatter(buf_l, [idxv], d - lo_v, mask=m)
            pos = pos + csum[15]
            return lax.cond(pos >= K_FLUSH, do_flush, no_flush, pos, flushed)

        def stage_loop(cb, carry):
            pltpu.sync_copy(src_hbm.at[pl.ds(cb * SCH, SCH)], stage_s)
            pltpu.sync_copy(dst_hbm.at[pl.ds(cb * SCH, SCH)], stage_d)
            return lax.fori_loop(0, SCH // 16, append_chunk, carry)

        pos, flushed = lax.fori_loop(0, N_EDGES // SCH, stage_loop,
                                     (jnp.int32(0), jnp.int32(0)))
        n_total = flushed + pos

        zeros16 = jnp.zeros((16,), jnp.int32)
        pad16 = jnp.full((16,), NPW, jnp.int32)
        for j in range(G // 16):
            buf_s[pl.ds(pos + j * 16, 16)] = zeros16
            buf_l[pl.ds(pos + j * 16, 16)] = pad16
        pos = pos + G
        pos, flushed = lax.cond(pos >= K_FLUSH, do_flush, no_flush,
                                pos, flushed)

        off = pl.multiple_of(w * ECAP + flushed, 8)
        pltpu.sync_copy(buf_s.at[pl.ds(0, K_FLUSH)],
                        bsrc_hbm.at[pl.ds(off, K_FLUSH)])
        pltpu.sync_copy(buf_l.at[pl.ds(0, K_FLUSH)],
                        bloc_hbm.at[pl.ds(off, K_FLUSH)])

        cnt_v[pl.ds(0, 16)] = jnp.full((16,), n_total, jnp.int32)
        pltpu.sync_copy(cnt_v, cnt_hbm.at[pl.ds(pl.multiple_of(w * 16, 8), 16)])

    return bin_edges


@functools.lru_cache(maxsize=None)
def _get_segmax(C):
    @functools.partial(
        pl.kernel,
        out_type=jax.ShapeDtypeStruct((NPAD, C), jnp.float32),
        mesh=_sc_mesh(),
        scratch_types=[
            pltpu.VMEM((NPW + 1, C), jnp.float32),
            pltpu.VMEM((G,), jnp.int32),
            pltpu.VMEM((G,), jnp.int32),
            pltpu.VMEM((G, C), jnp.float32),
            pltpu.VMEM((16,), jnp.int32),
            pltpu.SemaphoreType.DMA,
        ],
    )
    def seg_kernel(q_hbm, bsrc_hbm, bloc_hbm, cnt_hbm, s_hbm,
                   acc, idx, loc, rows, cnt_v, sem):
        w = _worker_id()
        lo = w * NPW
        pltpu.sync_copy(cnt_hbm.at[pl.ds(pl.multiple_of(w * 16, 8), 16)], cnt_v)
        n = cnt_v[pl.ds(0, 16)][0]
        nchunks = (n + (G - 1)) // G

        neg = jnp.full((16,), _NEG_INF, jnp.float32)

        def init_body(i, _):
            for j in range(C // 16):
                acc[i, pl.ds(j * 16, 16)] = neg
            return 0
        lax.fori_loop(0, NPW + 1, init_body, 0)

        def chunk_body(cb, _):
            base = pl.multiple_of(w * ECAP + cb * G, 8)
            pltpu.sync_copy(bsrc_hbm.at[pl.ds(base, G)], idx)
            pltpu.sync_copy(bloc_hbm.at[pl.ds(base, G)], loc)
            pltpu.async_copy(q_hbm.at[idx], rows, sem).wait()

            def group_body(g, _):
                locv = loc[pl.ds(g * 16, 16)]
                for t in range(16):
                    r = locv[t]
                    i = g * 16 + t
                    for j in range(C // 16):
                        sl = pl.ds(j * 16, 16)
                        acc[r, sl] = jnp.maximum(acc[r, sl], rows[i, sl])
                return 0
            lax.fori_loop(0, G // 16, group_body, 0)
            return 0
        lax.fori_loop(0, nchunks, chunk_body, 0)

        pltpu.sync_copy(acc.at[pl.ds(0, NPW)],
                        s_hbm.at[pl.ds(pl.multiple_of(lo, 8), NPW)])

    return seg_kernel


_TR = 1000


def _tc_first(x, A, bias, C, QW):
    cin = x.shape[1]

    def body(x_ref, a_ref, b_ref, p_ref, q_ref):
        r = jnp.dot(x_ref[...], a_ref[...],
                    preferred_element_type=jnp.float32) + b_ref[...]
        p_ref[...] = r[:, :C]
        q = r[:, C:]
        if QW > C:
            q = jnp.concatenate(
                [q, jnp.zeros((q.shape[0], QW - C), jnp.float32)], axis=1)
        q_ref[...] = q

    return pl.pallas_call(
        body,
        grid=(N_NODES // _TR,),
        in_specs=[
            pl.BlockSpec((_TR, cin), lambda i: (i, 0)),
            pl.BlockSpec((cin, 2 * C), lambda i: (0, 0)),
            pl.BlockSpec((1, 2 * C), lambda i: (0, 0)),
        ],
        out_specs=[
            pl.BlockSpec((_TR, C), lambda i: (i, 0)),
            pl.BlockSpec((_TR, QW), lambda i: (i, 0)),
        ],
        out_shape=[jax.ShapeDtypeStruct((N_NODES, C), jnp.float32),
                   jax.ShapeDtypeStruct((N_NODES, QW), jnp.float32)],
    )(x, A, bias)


def _tc_mid(p_prev, s_prev, A, bias, C):
    cin = p_prev.shape[1]

    def body(p_ref, s_ref, a_ref, b_ref, po_ref, qo_ref):
        xv = jnp.maximum(p_ref[...] + s_ref[...], 0.0)
        r = jnp.dot(xv, a_ref[...],
                    preferred_element_type=jnp.float32) + b_ref[...]
        po_ref[...] = r[:, :C]
        qo_ref[...] = r[:, C:]

    return pl.pallas_call(
        body,
        grid=(N_NODES // _TR,),
        in_specs=[
            pl.BlockSpec((_TR, cin), lambda i: (i, 0)),
            pl.BlockSpec((_TR, cin), lambda i: (i, 0)),
            pl.BlockSpec((cin, 2 * C), lambda i: (0, 0)),
            pl.BlockSpec((1, 2 * C), lambda i: (0, 0)),
        ],
        out_specs=[
            pl.BlockSpec((_TR, C), lambda i: (i, 0)),
            pl.BlockSpec((_TR, C), lambda i: (i, 0)),
        ],
        out_shape=[jax.ShapeDtypeStruct((N_NODES, C), jnp.float32)] * 2,
    )(p_prev, s_prev, A, bias)


def _tc_final(p3, s3a, s3b, x0, W4, b4, W5, b5):
    def body(p_ref, sa_ref, sb_ref, x0_ref, w4_ref, b4_ref, w5_ref, b5_ref,
             o_ref):
        s = jnp.concatenate([sa_ref[...], sb_ref[...]], axis=1)
        xv = jnp.maximum(p_ref[...] + s, 0.0)
        h = jnp.maximum(
            jnp.dot(xv, w4_ref[...], preferred_element_type=jnp.float32)
            + b4_ref[...], 0.0)
        o_ref[...] = (jnp.dot(h, w5_ref[...],
                              preferred_element_type=jnp.float32)
                      + b5_ref[...] + x0_ref[...])

    return pl.pallas_call(
        body,
        grid=(N_NODES // _TR,),
        in_specs=[
            pl.BlockSpec((_TR, 512), lambda i: (i, 0)),
            pl.BlockSpec((_TR, 256), lambda i: (i, 0)),
            pl.BlockSpec((_TR, 256), lambda i: (i, 0)),
            pl.BlockSpec((_TR, 3), lambda i: (i, 0)),
            pl.BlockSpec((512, 256), lambda i: (0, 0)),
            pl.BlockSpec((1, 256), lambda i: (0, 0)),
            pl.BlockSpec((256, 3), lambda i: (0, 0)),
            pl.BlockSpec((1, 3), lambda i: (0, 0)),
        ],
        out_specs=pl.BlockSpec((_TR, 3), lambda i: (i, 0)),
        out_shape=jax.ShapeDtypeStruct((N_NODES, 3), jnp.float32),
    )(p3, s3a, s3b, x0, W4, b4, W5, b5)


def _split_weights(W, b, cin):
    wa, wb = W[:cin], W[cin:]
    A = jnp.concatenate([wa - wb, wb], axis=1)
    bias = jnp.concatenate([b, jnp.zeros_like(b)])[None, :]
    return A, bias


def kernel(x, edge_index, W1, b1, W2, b2, W3, b3, W4, b4, W5, b5):
    src = edge_index[0]
    dst = edge_index[1]

    bsrc, bloc, counts = _get_bin_kernel()(src, dst)

    A1, bias1 = _split_weights(W1, b1, 3)
    A2, bias2 = _split_weights(W2, b2, 64)
    A3, bias3 = _split_weights(W3, b3, 128)

    P1, Q1 = _tc_first(x, A1, bias1, 64, 128)
    S1 = _get_segmax(128)(Q1, bsrc, bloc, counts)[:N_NODES, :64]

    P2, Q2 = _tc_mid(P1, S1, A2, bias2, 128)
    S2 = _get_segmax(128)(Q2, bsrc, bloc, counts)[:N_NODES]

    P3, Q3 = _tc_mid(P2, S2, A3, bias3, 512)
    S3a = _get_segmax(256)(Q3[:, :256], bsrc, bloc, counts)[:N_NODES]
    S3b = _get_segmax(256)(Q3[:, 256:], bsrc, bloc, counts)[:N_NODES]

    return _tc_final(P3, S3a, S3b, x, W4, b4[None, :], W5, b5[None, :])


# speedup vs baseline: 2.5949x; 1.2712x over previous
import functools

import jax
import jax.numpy as jnp
from jax import lax
from jax.experimental import pallas as pl
from jax.experimental.pallas import tpu as pltpu
from jax.experimental.pallas import tpu_sc as plsc

N_NODES = 10000
N_EDGES = 160000

NC = 2
NS = 16
NW = NC * NS
NPW = 320
NPAD = NW * NPW

K_FLUSH = 4096
G = 128
ECAP = N_EDGES + K_FLUSH + 256
SCH = 8000
BUFCAP = K_FLUSH + 192

_NEG_INF = float("-inf")


def _worker_id():
    return lax.axis_index("s") * NC + lax.axis_index("c")


def _sc_mesh():
    return plsc.VectorSubcoreMesh(
        core_axis_name="c", subcore_axis_name="s",
        num_cores=NC, num_subcores=NS)


@functools.lru_cache(maxsize=None)
def _get_bin_kernel():
    @functools.partial(
        pl.kernel,
        out_type=[
            jax.ShapeDtypeStruct((NW * ECAP,), jnp.int32),
            jax.ShapeDtypeStruct((NW * ECAP,), jnp.int32),
            jax.ShapeDtypeStruct((NW * 16,), jnp.int32),
        ],
        mesh=_sc_mesh(),
        scratch_types=[
            pltpu.VMEM((SCH,), jnp.int32),
            pltpu.VMEM((SCH,), jnp.int32),
            pltpu.VMEM((SCH,), jnp.int32),
            pltpu.VMEM((SCH,), jnp.int32),
            pltpu.VMEM((BUFCAP,), jnp.int32),
            pltpu.VMEM((BUFCAP,), jnp.int32),
            pltpu.VMEM((16,), jnp.int32),
            pltpu.SemaphoreType.DMA,
            pltpu.SemaphoreType.DMA,
        ],
        compiler_params=pltpu.CompilerParams(needs_layout_passes=False),
    )
    def bin_edges(src_hbm, dst_hbm, bsrc_hbm, bloc_hbm, cnt_hbm,
                  stage_sa, stage_da, stage_sb, stage_db,
                  buf_s, buf_l, cnt_v, sem_a, sem_b):
        w = _worker_id()
        lo = w * NPW

        def do_flush(pos, flushed):
            off = pl.multiple_of(w * ECAP + flushed, 8)
            pltpu.sync_copy(buf_s.at[pl.ds(0, K_FLUSH)],
                            bsrc_hbm.at[pl.ds(off, K_FLUSH)])
            pltpu.sync_copy(buf_l.at[pl.ds(0, K_FLUSH)],
                            bloc_hbm.at[pl.ds(off, K_FLUSH)])
            ts = buf_s[pl.ds(K_FLUSH, 16)]
            tl = buf_l[pl.ds(K_FLUSH, 16)]
            buf_s[pl.ds(0, 16)] = ts
            buf_l[pl.ds(0, 16)] = tl
            return pos - K_FLUSH, flushed + K_FLUSH

        def no_flush(pos, flushed):
            return pos, flushed

        lo_v = jnp.full((16,), lo, jnp.int32)
        hi_v = jnp.full((16,), lo + NPW, jnp.int32)
        zero_v = jnp.zeros((16,), jnp.int32)
        one_v = jnp.full((16,), 1, jnp.int32)

        def make_append(ss, dd):
            def append_chunk(i, carry):
                pos, flushed = carry
                d = dd[pl.ds(i * 16, 16)]
                s = ss[pl.ds(i * 16, 16)]
                m = (d >= lo_v) & (d < hi_v)
                csum = plsc.cumsum(jnp.where(m, one_v, zero_v))
                pos_v = jnp.full((16,), pos, jnp.int32)
                idxv = jnp.maximum(pos_v + csum - one_v, zero_v)
                plsc.store_scatter(buf_s, [idxv], s, mask=m)
                plsc.store_scatter(buf_l, [idxv], d - lo_v, mask=m)
                pos = pos + csum[15]
                return lax.cond(pos >= K_FLUSH, do_flush, no_flush,
                                pos, flushed)
            return append_chunk

        nb = N_EDGES // SCH
        bufs = [(stage_sa, stage_da, sem_a), (stage_sb, stage_db, sem_b)]

        def issue(cb, b):
            ss, dd, sem = bufs[b]
            pltpu.async_copy(src_hbm.at[pl.ds(cb * SCH, SCH)], ss, sem)
            pltpu.async_copy(dst_hbm.at[pl.ds(cb * SCH, SCH)], dd, sem)

        def drain(b):
            ss, dd, sem = bufs[b]
            pltpu.make_async_copy(src_hbm.at[pl.ds(0, SCH)], ss, sem).wait()
            pltpu.make_async_copy(src_hbm.at[pl.ds(0, SCH)], dd, sem).wait()

        issue(0, 0)
        carry = (jnp.int32(0), jnp.int32(0))
        for cb in range(nb):
            b = cb % 2
            drain(b)
            if cb + 1 < nb:
                issue(cb + 1, 1 - b)
            ss, dd, _ = bufs[b]
            carry = lax.fori_loop(0, SCH // 16, make_append(ss, dd), carry)
        pos, flushed = carry
        n_total = flushed + pos

        zeros16 = jnp.zeros((16,), jnp.int32)
        pad16 = jnp.full((16,), NPW, jnp.int32)
        for j in range(G // 16):
            buf_s[pl.ds(pos + j * 16, 16)] = zeros16
            buf_l[pl.ds(pos + j * 16, 16)] = pad16
        pos = pos + G
        pos, flushed = lax.cond(pos >= K_FLUSH, do_flush, no_flush,
                                pos, flushed)

        off = pl.multiple_of(w * ECAP + flushed, 8)
        pltpu.sync_copy(buf_s.at[pl.ds(0, K_FLUSH)],
                        bsrc_hbm.at[pl.ds(off, K_FLUSH)])
        pltpu.sync_copy(buf_l.at[pl.ds(0, K_FLUSH)],
                        bloc_hbm.at[pl.ds(off, K_FLUSH)])

        cnt_v[pl.ds(0, 16)] = jnp.full((16,), n_total, jnp.int32)
        pltpu.sync_copy(cnt_v, cnt_hbm.at[pl.ds(pl.multiple_of(w * 16, 8), 16)])

    return bin_edges


IB = 4096


@functools.lru_cache(maxsize=None)
def _get_segmax(C):
    g = 64 if C > 128 else 128
    cpb = IB // g

    @functools.partial(
        pl.kernel,
        out_type=jax.ShapeDtypeStruct((NPAD, C), jnp.float32),
        mesh=_sc_mesh(),
        scratch_types=[
            pltpu.VMEM((NPW + 1, C), jnp.float32),
            pltpu.VMEM((g, C), jnp.float32),
            pltpu.VMEM((g, C), jnp.float32),
            pltpu.VMEM((IB,), jnp.int32),
            pltpu.VMEM((IB,), jnp.int32),
            pltpu.VMEM((16,), jnp.int32),
            pltpu.SemaphoreType.DMA,
            pltpu.SemaphoreType.DMA,
        ],
    )
    def seg_kernel(q_hbm, bsrc_hbm, bloc_hbm, cnt_hbm, s_hbm,
                   acc, rows_a, rows_b, ibuf_s, ibuf_l, cnt_v, sem_a, sem_b):
        w = _worker_id()
        lo = w * NPW
        pltpu.sync_copy(cnt_hbm.at[pl.ds(pl.multiple_of(w * 16, 8), 16)],
                        cnt_v)
        n = cnt_v[pl.ds(0, 16)][0]
        nchunks = (n + (g - 1)) // g
        nblocks = (nchunks + (cpb - 1)) // cpb

        neg = jnp.full((16,), _NEG_INF, jnp.float32)

        def init_body(i, _):
            for j in range(C // 16):
                acc[i, pl.ds(j * 16, 16)] = neg
            return 0
        lax.fori_loop(0, NPW + 1, init_body, 0)

        def compute(rows, c):
            def group_body(gi, _):
                locv = ibuf_l[pl.ds(c * g + gi * 16, 16)]
                for t in range(16):
                    r = locv[t]
                    i = gi * 16 + t
                    for j in range(C // 16):
                        sl = pl.ds(j * 16, 16)
                        acc[r, sl] = jnp.maximum(acc[r, sl], rows[i, sl])
                return 0
            lax.fori_loop(0, g // 16, group_body, 0)

        def gather(c, rows, sem):
            pltpu.async_copy(q_hbm.at[ibuf_s.at[pl.ds(c * g, g)]], rows, sem)

        def wait(rows, sem):
            pltpu.make_async_copy(q_hbm.at[ibuf_s.at[pl.ds(0, g)]],
                                  rows, sem).wait()

        def block_body(ib, _):
            boff = pl.multiple_of(w * ECAP + ib * IB, 8)
            pltpu.sync_copy(bsrc_hbm.at[pl.ds(boff, IB)], ibuf_s)
            pltpu.sync_copy(bloc_hbm.at[pl.ds(boff, IB)], ibuf_l)
            ch = jnp.minimum(nchunks - ib * cpb, cpb)
            gather(0, rows_a, sem_a)

            def pair_body(p, _):
                c0 = 2 * p
                wait(rows_a, sem_a)

                @pl.when(c0 + 1 < ch)
                def _():
                    gather(c0 + 1, rows_b, sem_b)
                compute(rows_a, c0)

                @pl.when(c0 + 1 < ch)
                def _():
                    wait(rows_b, sem_b)

                    @pl.when(c0 + 2 < ch)
                    def _():
                        gather(c0 + 2, rows_a, sem_a)
                    compute(rows_b, c0 + 1)
                return 0
            lax.fori_loop(0, (ch + 1) // 2, pair_body, 0)
            return 0
        lax.fori_loop(0, nblocks, block_body, 0)

        pltpu.sync_copy(acc.at[pl.ds(0, NPW)],
                        s_hbm.at[pl.ds(pl.multiple_of(lo, 8), NPW)])

    return seg_kernel


_TR = 1000


def _tc_first(x, A, bias, C, QW):
    cin = x.shape[1]

    def body(x_ref, a_ref, b_ref, p_ref, q_ref):
        r = jnp.dot(x_ref[...], a_ref[...],
                    preferred_element_type=jnp.float32) + b_ref[...]
        p_ref[...] = r[:, :C]
        q = r[:, C:]
        if QW > C:
            q = jnp.concatenate(
                [q, jnp.zeros((q.shape[0], QW - C), jnp.float32)], axis=1)
        q_ref[...] = q

    return pl.pallas_call(
        body,
        grid=(N_NODES // _TR,),
        in_specs=[
            pl.BlockSpec((_TR, cin), lambda i: (i, 0)),
            pl.BlockSpec((cin, 2 * C), lambda i: (0, 0)),
            pl.BlockSpec((1, 2 * C), lambda i: (0, 0)),
        ],
        out_specs=[
            pl.BlockSpec((_TR, C), lambda i: (i, 0)),
            pl.BlockSpec((_TR, QW), lambda i: (i, 0)),
        ],
        out_shape=[jax.ShapeDtypeStruct((N_NODES, C), jnp.float32),
                   jax.ShapeDtypeStruct((N_NODES, QW), jnp.float32)],
    )(x, A, bias)


def _tc_mid(p_prev, s_prev, A, bias, C):
    cin = p_prev.shape[1]

    def body(p_ref, s_ref, a_ref, b_ref, po_ref, qo_ref):
        xv = jnp.maximum(p_ref[...] + s_ref[...], 0.0)
        r = jnp.dot(xv, a_ref[...],
                    preferred_element_type=jnp.float32) + b_ref[...]
        po_ref[...] = r[:, :C]
        qo_ref[...] = r[:, C:]

    return pl.pallas_call(
        body,
        grid=(N_NODES // _TR,),
        in_specs=[
            pl.BlockSpec((_TR, cin), lambda i: (i, 0)),
            pl.BlockSpec((_TR, cin), lambda i: (i, 0)),
            pl.BlockSpec((cin, 2 * C), lambda i: (0, 0)),
            pl.BlockSpec((1, 2 * C), lambda i: (0, 0)),
        ],
        out_specs=[
            pl.BlockSpec((_TR, C), lambda i: (i, 0)),
            pl.BlockSpec((_TR, C), lambda i: (i, 0)),
        ],
        out_shape=[jax.ShapeDtypeStruct((N_NODES, C), jnp.float32)] * 2,
    )(p_prev, s_prev, A, bias)


def _tc_final(p3, s3a, s3b, x0, W4, b4, W5, b5):
    def body(p_ref, sa_ref, sb_ref, x0_ref, w4_ref, b4_ref, w5_ref, b5_ref,
             o_ref):
        s = jnp.concatenate([sa_ref[...], sb_ref[...]], axis=1)
        xv = jnp.maximum(p_ref[...] + s, 0.0)
        h = jnp.maximum(
            jnp.dot(xv, w4_ref[...], preferred_element_type=jnp.float32)
            + b4_ref[...], 0.0)
        o_ref[...] = (jnp.dot(h, w5_ref[...],
                              preferred_element_type=jnp.float32)
                      + b5_ref[...] + x0_ref[...])

    return pl.pallas_call(
        body,
        grid=(N_NODES // _TR,),
        in_specs=[
            pl.BlockSpec((_TR, 512), lambda i: (i, 0)),
            pl.BlockSpec((_TR, 256), lambda i: (i, 0)),
            pl.BlockSpec((_TR, 256), lambda i: (i, 0)),
            pl.BlockSpec((_TR, 3), lambda i: (i, 0)),
            pl.BlockSpec((512, 256), lambda i: (0, 0)),
            pl.BlockSpec((1, 256), lambda i: (0, 0)),
            pl.BlockSpec((256, 3), lambda i: (0, 0)),
            pl.BlockSpec((1, 3), lambda i: (0, 0)),
        ],
        out_specs=pl.BlockSpec((_TR, 3), lambda i: (i, 0)),
        out_shape=jax.ShapeDtypeStruct((N_NODES, 3), jnp.float32),
    )(p3, s3a, s3b, x0, W4, b4, W5, b5)


def _split_weights(W, b, cin):
    wa, wb = W[:cin], W[cin:]
    A = jnp.concatenate([wa - wb, wb], axis=1)
    bias = jnp.concatenate([b, jnp.zeros_like(b)])[None, :]
    return A, bias


def kernel(x, edge_index, W1, b1, W2, b2, W3, b3, W4, b4, W5, b5):
    src = edge_index[0]
    dst = edge_index[1]

    bsrc, bloc, counts = _get_bin_kernel()(src, dst)

    A1, bias1 = _split_weights(W1, b1, 3)
    A2, bias2 = _split_weights(W2, b2, 64)
    A3, bias3 = _split_weights(W3, b3, 128)

    P1, Q1 = _tc_first(x, A1, bias1, 64, 128)
    S1 = _get_segmax(128)(Q1, bsrc, bloc, counts)[:N_NODES, :64]

    P2, Q2 = _tc_mid(P1, S1, A2, bias2, 128)
    S2 = _get_segmax(128)(Q2, bsrc, bloc, counts)[:N_NODES]

    P3, Q3 = _tc_mid(P2, S2, A3, bias3, 512)
    S3a = _get_segmax(256)(Q3[:, :256], bsrc, bloc, counts)[:N_NODES]
    S3b = _get_segmax(256)(Q3[:, 256:], bsrc, bloc, counts)[:N_NODES]

    return _tc_final(P3, S3a, S3b, x, W4, b4[None, :], W5, b5[None, :])
